# Initial kernel scaffold; baseline (speedup 1.0000x reference)
import jax
import jax.numpy as jnp
from jax.experimental import pallas as pl


def kernel(adjacency, graph_indicator, eeg, eye, au, W1, W2, w_score):
    raise NotImplementedError("write your pallas kernel here")


# speedup vs baseline: 3.9113x; 3.9113x over previous
import functools

import numpy as np
import jax
import jax.numpy as jnp
from jax import lax
from jax.experimental import pallas as pl
from jax.experimental.pallas import tpu as pltpu
from jax.experimental.pallas import tpu_sc as plsc

N = 10000
E = 320000
D_IN = 128
DH = 64
G = 64
K = 5000

NW = 32
C = 128
CH = 80
EPAD = NW * CH * C
NPAD = 10112
RPW = NPAD // 16
NBUF = 4
INT_MIN = np.int32(-(2 ** 31))


def _sc_body(val, srcr, dstr, zer, out, idx_s, idx_d, ring, acc, gsem,
             zer8=None, ones8=None, deg=None, ones_v=None, dacc=None):
    c = lax.axis_index("c")
    s = lax.axis_index("s")
    wid = s * 2 + c
    pltpu.sync_copy(srcr.at[wid], idx_s)
    pltpu.sync_copy(dstr.at[wid], idx_d)
    pltpu.sync_copy(zer.at[pl.ds(s * RPW, RPW)], acc.at[pl.ds(s * RPW, RPW)])
    if dacc is not None:
        pltpu.sync_copy(zer8.at[pl.ds(s * RPW, RPW)], dacc.at[pl.ds(s * RPW, RPW)])
        pltpu.sync_copy(ones8, ones_v)
    plsc.subcore_barrier()

    for b in range(NBUF):
        pltpu.async_copy(val.at[idx_s.at[b]], ring.at[b], gsem.at[b])

    @pl.loop(0, CH, step=NBUF)
    def _outer(jo):
        for b in range(NBUF):
            j = jo + b
            pltpu.make_async_copy(val.at[idx_s.at[j]], ring.at[b], gsem.at[b]).wait()
            pltpu.sync_copy(ring.at[b], acc.at[idx_d.at[j]], add=True)
            if dacc is not None:
                pltpu.sync_copy(ones_v, dacc.at[idx_d.at[j]], add=True)
            nj = j + NBUF

            @pl.when(nj < CH)
            def _fire():
                pltpu.async_copy(val.at[idx_s.at[nj]], ring.at[b], gsem.at[b])

    plsc.subcore_barrier()
    pltpu.sync_copy(acc.at[pl.ds(s * RPW, RPW)], out.at[c, pl.ds(s * RPW, RPW)])
    if dacc is not None:
        pltpu.sync_copy(dacc.at[pl.ds(s * RPW, RPW)], deg.at[c, pl.ds(s * RPW, RPW)])


@functools.cache
def _make_sc_segsum(D, with_deg):
    mesh = plsc.VectorSubcoreMesh(core_axis_name="c", subcore_axis_name="s")
    out_type = jax.ShapeDtypeStruct((2, NPAD, D), jnp.float32)
    scratch = [
        pltpu.VMEM((CH, C), jnp.int32),
        pltpu.VMEM((CH, C), jnp.int32),
        pltpu.VMEM((NBUF, C, D), jnp.float32),
        pltpu.VMEM_SHARED((NPAD, D), jnp.float32),
        pltpu.SemaphoreType.DMA((NBUF,)),
    ]
    if with_deg:
        out_type = (out_type, jax.ShapeDtypeStruct((2, NPAD, 8), jnp.float32))
        scratch += [
            pltpu.VMEM((C, 8), jnp.float32),
            pltpu.VMEM_SHARED((NPAD, 8), jnp.float32),
        ]

        def body(val, srcr, dstr, zer, zer8, ones8, out, deg,
                 idx_s, idx_d, ring, acc, gsem, ones_v, dacc):
            _sc_body(val, srcr, dstr, zer, out, idx_s, idx_d, ring, acc, gsem,
                     zer8=zer8, ones8=ones8, deg=deg, ones_v=ones_v, dacc=dacc)
    else:
        def body(val, srcr, dstr, zer, out, idx_s, idx_d, ring, acc, gsem):
            _sc_body(val, srcr, dstr, zer, out, idx_s, idx_d, ring, acc, gsem)

    return pl.kernel(body, out_type=out_type, mesh=mesh, scratch_types=scratch,
                     compiler_params=pltpu.CompilerParams(use_tc_tiling_on_sc=False))


def _layer(a0, a1, d0, d1, w):
    def body(a0r, a1r, d0r, d1r, wr, o_ref):
        degc = jnp.maximum((d0r[...] + d1r[...])[:, 0:1], 1.0)
        agg = (a0r[...] + a1r[...]) / degc
        o_ref[...] = jnp.maximum(
            jnp.dot(agg, wr[...], preferred_element_type=jnp.float32), 0.0)

    return pl.pallas_call(
        body,
        out_shape=jax.ShapeDtypeStruct((NPAD, w.shape[1]), jnp.float32),
    )(a0, a1, d0, d1, w)


def _layer_cat(a0, a1, b0, b1, d0, d1, w):
    def body(a0r, a1r, b0r, b1r, d0r, d1r, wr, o_ref):
        degc = jnp.maximum((d0r[...] + d1r[...])[:, 0:1], 1.0)
        agg = jnp.concatenate(
            [(a0r[...] + a1r[...]) / degc, (b0r[...] + b1r[...]) / degc], axis=1)
        o_ref[...] = jnp.maximum(
            jnp.dot(agg, wr[...], preferred_element_type=jnp.float32), 0.0)

    return pl.pallas_call(
        body,
        out_shape=jax.ShapeDtypeStruct((NPAD, w.shape[1]), jnp.float32),
    )(a0, a1, b0, b1, d0, d1, w)


def _scorer(a0, a1, d0, d1, wpad):
    def body(a0r, a1r, d0r, d1r, wr, w_ref):
        degc = jnp.maximum((d0r[...] + d1r[...])[:, 0:1], 1.0)
        agg = (a0r[...] + a1r[...]) / degc
        sc8 = jnp.tanh(jnp.dot(agg, wr[...], preferred_element_type=jnp.float32))
        row = lax.broadcasted_iota(jnp.int32, (NPAD, 8), 0)
        col = lax.broadcasted_iota(jnp.int32, (NPAD, 8), 1)
        valid = (row < N) & (col == 0)
        bits = lax.bitcast_convert_type(sc8, jnp.int32)
        skey = jnp.where(bits >= 0, bits, jnp.invert(bits) ^ INT_MIN)
        skey = jnp.where(valid, skey, INT_MIN)

        def sbody(t, pref):
            cand = pref | (jnp.int32(1) << (31 - t))
            cnt = jnp.sum((skey >= (cand ^ INT_MIN)).astype(jnp.int32))
            return jnp.where(cnt >= K, cand, pref)

        pref = lax.fori_loop(0, 32, sbody, jnp.int32(0))
        vs = pref ^ INT_MIN
        cgt = jnp.sum((skey > vs).astype(jnp.int32))
        m = K - cgt
        tie = skey == vs

        def s2body(t, ans):
            cand = ans | (jnp.int32(1) << (13 - t))
            cnt = jnp.sum((tie & (row < cand)).astype(jnp.int32))
            return jnp.where(cnt < m, cand, ans)

        ans = lax.fori_loop(0, 14, s2body, jnp.int32(0))
        sel = (skey > vs) | (tie & (row <= ans))
        w_ref[...] = jnp.where(sel, sc8, 0.0)

    return pl.pallas_call(
        body,
        out_shape=jax.ShapeDtypeStruct((NPAD, 8), jnp.float32),
    )(a0, a1, d0, d1, wpad)


def _readout(w8, gib, h2):
    def body(w_ref, gi_ref, h2_ref, o_ref):
        hm = h2_ref[...] * w_ref[...][:, 0:1]
        giota = lax.broadcasted_iota(jnp.int32, (1, G), 1)
        oh = (gi_ref[...] == giota).astype(jnp.float32)
        o_ref[...] = lax.dot_general(
            oh, hm, (((0,), (0,)), ((), ())),
            preferred_element_type=jnp.float32)

    return pl.pallas_call(
        body,
        out_shape=jax.ShapeDtypeStruct((G, DH), jnp.float32),
    )(w8, gib, h2)


def kernel(adjacency, graph_indicator, eeg, eye, au, W1, W2, w_score):
    src = adjacency[0]
    dst = adjacency[1]
    x = jnp.pad(eeg.reshape(-1, D_IN), ((0, NPAD - N), (0, 0)))
    xa = x[:, :DH]
    xb = x[:, DH:]

    pad = EPAD - E
    srcr = jnp.concatenate([src, jnp.zeros((pad,), jnp.int32)]).reshape(NW, CH, C)
    dstr = jnp.concatenate([dst, jnp.full((pad,), N, jnp.int32)]).reshape(NW, CH, C)
    zer64 = jnp.zeros((NPAD, DH), jnp.float32)
    zer8 = jnp.zeros((NPAD, 8), jnp.float32)
    ones8 = jnp.ones((C, 8), jnp.float32)

    sum1a, degp = _make_sc_segsum(DH, True)(xa, srcr, dstr, zer64, zer8, ones8)
    sum1b = _make_sc_segsum(DH, False)(xb, srcr, dstr, zer64)
    h1 = _layer_cat(sum1a[0], sum1a[1], sum1b[0], sum1b[1],
                    degp[0], degp[1], W1)
    sum2 = _make_sc_segsum(DH, False)(h1, srcr, dstr, zer64)
    h2 = _layer(sum2[0], sum2[1], degp[0], degp[1], W2)
    sum3 = _make_sc_segsum(DH, False)(h2, srcr, dstr, zer64)
    wpad = jnp.pad(w_score, ((0, 0), (0, 7)))
    w8 = _scorer(sum3[0], sum3[1], degp[0], degp[1], wpad)
    gip = jnp.pad(graph_indicator, (0, NPAD - N), constant_values=G)
    gib = jnp.broadcast_to(gip[:, None], (NPAD, G))
    eeg_out = _readout(w8, gib, h2)
    return (eeg_out, eye, au)


# speedup vs baseline: 3.9375x; 1.0067x over previous
import functools

import numpy as np
import jax
import jax.numpy as jnp
from jax import lax
from jax.experimental import pallas as pl
from jax.experimental.pallas import tpu as pltpu
from jax.experimental.pallas import tpu_sc as plsc

N = 10000
E = 320000
D_IN = 128
DH = 64
G = 64
K = 5000

NW = 32
C = 128
CH = 80
EPAD = NW * CH * C
NPAD = 10112
RPW = NPAD // 16
INT_MIN = np.int32(-(2 ** 31))


def _sc_body(ring_n, fire, val, srcr, dstr, zer, out, idx_s, idx_d, ring, acc,
             gsem, ssem, zer8=None, ones8=None, deg=None, ones_v=None,
             dacc=None, dsem=None):
    c = lax.axis_index("c")
    s = lax.axis_index("s")
    wid = s * 2 + c
    pltpu.sync_copy(srcr.at[wid], idx_s)
    pltpu.sync_copy(dstr.at[wid], idx_d)
    for b in range(fire):
        pltpu.async_copy(val.at[idx_s.at[b]], ring.at[b], gsem.at[b])
    pltpu.sync_copy(zer.at[pl.ds(s * RPW, RPW)], acc.at[pl.ds(s * RPW, RPW)])
    if dacc is not None:
        pltpu.sync_copy(zer8.at[pl.ds(s * RPW, RPW)], dacc.at[pl.ds(s * RPW, RPW)])
        pltpu.sync_copy(ones8, ones_v)
    plsc.subcore_barrier()

    @pl.loop(0, CH, step=ring_n)
    def _outer(jo):
        for b in range(ring_n):
            j = jo + b
            pltpu.make_async_copy(val.at[idx_s.at[j]], ring.at[b], gsem.at[b]).wait()
            pltpu.async_copy(ring.at[b], acc.at[idx_d.at[j]], ssem.at[b], add=True)
            if dacc is not None:
                pltpu.async_copy(ones_v, dacc.at[idx_d.at[j]], dsem.at[b], add=True)
            fj = j + fire
            bf = (b + fire) % ring_n

            @pl.when((fj >= ring_n) & (fj < CH))
            def _drain():
                pj = fj - ring_n
                pltpu.make_async_copy(ring.at[bf], acc.at[idx_d.at[pj]],
                                      ssem.at[bf]).wait()
                if dacc is not None:
                    pltpu.make_async_copy(ones_v, dacc.at[idx_d.at[pj]],
                                          dsem.at[bf]).wait()

            @pl.when(fj < CH)
            def _fire():
                pltpu.async_copy(val.at[idx_s.at[fj]], ring.at[bf], gsem.at[bf])

    for q in range(CH - ring_n, CH):
        bq = q % ring_n
        pltpu.make_async_copy(ring.at[bq], acc.at[idx_d.at[q]], ssem.at[bq]).wait()
        if dacc is not None:
            pltpu.make_async_copy(ones_v, dacc.at[idx_d.at[q]], dsem.at[bq]).wait()

    plsc.subcore_barrier()
    pltpu.sync_copy(acc.at[pl.ds(s * RPW, RPW)], out.at[c, pl.ds(s * RPW, RPW)])
    if dacc is not None:
        pltpu.sync_copy(dacc.at[pl.ds(s * RPW, RPW)], deg.at[c, pl.ds(s * RPW, RPW)])


@functools.cache
def _make_sc_segsum(D, with_deg):
    mesh = plsc.VectorSubcoreMesh(core_axis_name="c", subcore_axis_name="s")
    ring_n = 4 if with_deg else 8
    fire = ring_n // 2
    out_type = jax.ShapeDtypeStruct((2, NPAD, D), jnp.float32)
    scratch = [
        pltpu.VMEM((CH, C), jnp.int32),
        pltpu.VMEM((CH, C), jnp.int32),
        pltpu.VMEM((ring_n, C, D), jnp.float32),
        pltpu.VMEM_SHARED((NPAD, D), jnp.float32),
        pltpu.SemaphoreType.DMA((ring_n,)),
        pltpu.SemaphoreType.DMA((ring_n,)),
    ]
    if with_deg:
        out_type = (out_type, jax.ShapeDtypeStruct((2, NPAD, 8), jnp.float32))
        scratch += [
            pltpu.VMEM((C, 8), jnp.float32),
            pltpu.VMEM_SHARED((NPAD, 8), jnp.float32),
            pltpu.SemaphoreType.DMA((ring_n,)),
        ]

        def body(val, srcr, dstr, zer, zer8, ones8, out, deg,
                 idx_s, idx_d, ring, acc, gsem, ssem, ones_v, dacc, dsem):
            _sc_body(ring_n, fire, val, srcr, dstr, zer, out, idx_s, idx_d,
                     ring, acc, gsem, ssem, zer8=zer8, ones8=ones8, deg=deg,
                     ones_v=ones_v, dacc=dacc, dsem=dsem)
    else:
        def body(val, srcr, dstr, zer, out, idx_s, idx_d, ring, acc, gsem, ssem):
            _sc_body(ring_n, fire, val, srcr, dstr, zer, out, idx_s, idx_d,
                     ring, acc, gsem, ssem)

    return pl.kernel(body, out_type=out_type, mesh=mesh, scratch_types=scratch,
                     compiler_params=pltpu.CompilerParams(use_tc_tiling_on_sc=False))


def _layer(a0, a1, d0, d1, w):
    def body(a0r, a1r, d0r, d1r, wr, o_ref):
        degc = jnp.maximum((d0r[...] + d1r[...])[:, 0:1], 1.0)
        agg = (a0r[...] + a1r[...]) / degc
        o_ref[...] = jnp.maximum(
            jnp.dot(agg, wr[...], preferred_element_type=jnp.float32), 0.0)

    return pl.pallas_call(
        body,
        out_shape=jax.ShapeDtypeStruct((NPAD, w.shape[1]), jnp.float32),
    )(a0, a1, d0, d1, w)


def _layer_cat(a0, a1, b0, b1, d0, d1, w):
    def body(a0r, a1r, b0r, b1r, d0r, d1r, wr, o_ref):
        degc = jnp.maximum((d0r[...] + d1r[...])[:, 0:1], 1.0)
        agg = jnp.concatenate(
            [(a0r[...] + a1r[...]) / degc, (b0r[...] + b1r[...]) / degc], axis=1)
        o_ref[...] = jnp.maximum(
            jnp.dot(agg, wr[...], preferred_element_type=jnp.float32), 0.0)

    return pl.pallas_call(
        body,
        out_shape=jax.ShapeDtypeStruct((NPAD, w.shape[1]), jnp.float32),
    )(a0, a1, b0, b1, d0, d1, w)


def _scorer(a0, a1, d0, d1, wpad):
    def body(a0r, a1r, d0r, d1r, wr, w_ref):
        degc = jnp.maximum((d0r[...] + d1r[...])[:, 0:1], 1.0)
        agg = (a0r[...] + a1r[...]) / degc
        sc8 = jnp.tanh(jnp.dot(agg, wr[...], preferred_element_type=jnp.float32))
        row = lax.broadcasted_iota(jnp.int32, (NPAD, 8), 0)
        col = lax.broadcasted_iota(jnp.int32, (NPAD, 8), 1)
        valid = (row < N) & (col == 0)
        bits = lax.bitcast_convert_type(sc8, jnp.int32)
        skey = jnp.where(bits >= 0, bits, jnp.invert(bits) ^ INT_MIN)
        skey = jnp.where(valid, skey, INT_MIN)

        def sbody(t, pref):
            cand = pref | (jnp.int32(1) << (31 - t))
            cnt = jnp.sum((skey >= (cand ^ INT_MIN)).astype(jnp.int32))
            return jnp.where(cnt >= K, cand, pref)

        pref = lax.fori_loop(0, 32, sbody, jnp.int32(0))
        vs = pref ^ INT_MIN
        cgt = jnp.sum((skey > vs).astype(jnp.int32))
        m = K - cgt
        tie = skey == vs

        def s2body(t, ans):
            cand = ans | (jnp.int32(1) << (13 - t))
            cnt = jnp.sum((tie & (row < cand)).astype(jnp.int32))
            return jnp.where(cnt < m, cand, ans)

        ans = lax.fori_loop(0, 14, s2body, jnp.int32(0))
        sel = (skey > vs) | (tie & (row <= ans))
        w_ref[...] = jnp.where(sel, sc8, 0.0)

    return pl.pallas_call(
        body,
        out_shape=jax.ShapeDtypeStruct((NPAD, 8), jnp.float32),
    )(a0, a1, d0, d1, wpad)


def _readout(w8, gib, h2):
    def body(w_ref, gi_ref, h2_ref, o_ref):
        hm = h2_ref[...] * w_ref[...][:, 0:1]
        giota = lax.broadcasted_iota(jnp.int32, (1, G), 1)
        oh = (gi_ref[...] == giota).astype(jnp.float32)
        o_ref[...] = lax.dot_general(
            oh, hm, (((0,), (0,)), ((), ())),
            preferred_element_type=jnp.float32)

    return pl.pallas_call(
        body,
        out_shape=jax.ShapeDtypeStruct((G, DH), jnp.float32),
    )(w8, gib, h2)


def kernel(adjacency, graph_indicator, eeg, eye, au, W1, W2, w_score):
    src = adjacency[0]
    dst = adjacency[1]
    x = jnp.pad(eeg.reshape(-1, D_IN), ((0, NPAD - N), (0, 0)))
    xa = x[:, :DH]
    xb = x[:, DH:]

    pad = EPAD - E
    srcr = jnp.concatenate([src, jnp.zeros((pad,), jnp.int32)]).reshape(NW, CH, C)
    dstr = jnp.concatenate([dst, jnp.full((pad,), N, jnp.int32)]).reshape(NW, CH, C)
    zer64 = jnp.zeros((NPAD, DH), jnp.float32)
    zer8 = jnp.zeros((NPAD, 8), jnp.float32)
    ones8 = jnp.ones((C, 8), jnp.float32)

    sum1a, degp = _make_sc_segsum(DH, True)(xa, srcr, dstr, zer64, zer8, ones8)
    sum1b = _make_sc_segsum(DH, False)(xb, srcr, dstr, zer64)
    h1 = _layer_cat(sum1a[0], sum1a[1], sum1b[0], sum1b[1],
                    degp[0], degp[1], W1)
    sum2 = _make_sc_segsum(DH, False)(h1, srcr, dstr, zer64)
    h2 = _layer(sum2[0], sum2[1], degp[0], degp[1], W2)
    sum3 = _make_sc_segsum(DH, False)(h2, srcr, dstr, zer64)
    wpad = jnp.pad(w_score, ((0, 0), (0, 7)))
    w8 = _scorer(sum3[0], sum3[1], degp[0], degp[1], wpad)
    gip = jnp.pad(graph_indicator, (0, NPAD - N), constant_values=G)
    gib = jnp.broadcast_to(gip[:, None], (NPAD, G))
    eeg_out = _readout(w8, gib, h2)
    return (eeg_out, eye, au)


# speedup vs baseline: 4.1596x; 1.0564x over previous
import functools

import numpy as np
import jax
import jax.numpy as jnp
from jax import lax
from jax.experimental import pallas as pl
from jax.experimental.pallas import tpu as pltpu
from jax.experimental.pallas import tpu_sc as plsc

N = 10000
E = 320000
D_IN = 128
DH = 64
G = 64
K = 5000

NW = 32
C = 128
CH = 80
EPAD = NW * CH * C
NPAD = 10112
RPW = NPAD // 16
INT_MIN = np.int32(-(2 ** 31))


def _sc_body(ring_n, fire, val, srcr, dstr, zer, out, idx_s, idx_d, ring, acc,
             gsem, ssem, zer8=None, ones8=None, deg=None, ones_v=None,
             dacc=None, dsem=None):
    c = lax.axis_index("c")
    s = lax.axis_index("s")
    wid = s * 2 + c
    pltpu.sync_copy(srcr.at[wid], idx_s)
    pltpu.sync_copy(dstr.at[wid], idx_d)
    for b in range(fire):
        pltpu.async_copy(val.at[idx_s.at[b]], ring.at[b], gsem.at[b])
    pltpu.sync_copy(zer.at[pl.ds(s * RPW, RPW)], acc.at[pl.ds(s * RPW, RPW)])
    if dacc is not None:
        pltpu.sync_copy(zer8.at[pl.ds(s * RPW, RPW)], dacc.at[pl.ds(s * RPW, RPW)])
        pltpu.sync_copy(ones8, ones_v)
    plsc.subcore_barrier()

    @pl.loop(0, CH, step=ring_n)
    def _outer(jo):
        for b in range(ring_n):
            j = jo + b
            pltpu.make_async_copy(val.at[idx_s.at[j]], ring.at[b], gsem.at[b]).wait()
            pltpu.async_copy(ring.at[b], acc.at[idx_d.at[j]], ssem.at[b], add=True)
            if dacc is not None:
                pltpu.async_copy(ones_v, dacc.at[idx_d.at[j]], dsem.at[b], add=True)
            fj = j + fire
            bf = (b + fire) % ring_n

            @pl.when((fj >= ring_n) & (fj < CH))
            def _drain():
                pj = fj - ring_n
                pltpu.make_async_copy(ring.at[bf], acc.at[idx_d.at[pj]],
                                      ssem.at[bf]).wait()
                if dacc is not None:
                    pltpu.make_async_copy(ones_v, dacc.at[idx_d.at[pj]],
                                          dsem.at[bf]).wait()

            @pl.when(fj < CH)
            def _fire():
                pltpu.async_copy(val.at[idx_s.at[fj]], ring.at[bf], gsem.at[bf])

    for q in range(CH - ring_n, CH):
        bq = q % ring_n
        pltpu.make_async_copy(ring.at[bq], acc.at[idx_d.at[q]], ssem.at[bq]).wait()
        if dacc is not None:
            pltpu.make_async_copy(ones_v, dacc.at[idx_d.at[q]], dsem.at[bq]).wait()

    plsc.subcore_barrier()
    pltpu.sync_copy(acc.at[pl.ds(s * RPW, RPW)], out.at[c, pl.ds(s * RPW, RPW)])
    if dacc is not None:
        pltpu.sync_copy(dacc.at[pl.ds(s * RPW, RPW)], deg.at[c, pl.ds(s * RPW, RPW)])


@functools.cache
def _make_sc_segsum(D, with_deg):
    mesh = plsc.VectorSubcoreMesh(core_axis_name="c", subcore_axis_name="s")
    ring_n = 5 if with_deg else 8
    fire = ring_n // 2
    out_type = jax.ShapeDtypeStruct((2, NPAD, D), jnp.float32)
    scratch = [
        pltpu.VMEM((CH, C), jnp.int32),
        pltpu.VMEM((CH, C), jnp.int32),
        pltpu.VMEM((ring_n, C, D), jnp.float32),
        pltpu.VMEM_SHARED((NPAD, D), jnp.float32),
        pltpu.SemaphoreType.DMA((ring_n,)),
        pltpu.SemaphoreType.DMA((ring_n,)),
    ]
    if with_deg:
        out_type = (out_type, jax.ShapeDtypeStruct((2, NPAD, 8), jnp.float32))
        scratch += [
            pltpu.VMEM((C, 8), jnp.float32),
            pltpu.VMEM_SHARED((NPAD, 8), jnp.float32),
            pltpu.SemaphoreType.DMA((ring_n,)),
        ]

        def body(val, srcr, dstr, zer, zer8, ones8, out, deg,
                 idx_s, idx_d, ring, acc, gsem, ssem, ones_v, dacc, dsem):
            _sc_body(ring_n, fire, val, srcr, dstr, zer, out, idx_s, idx_d,
                     ring, acc, gsem, ssem, zer8=zer8, ones8=ones8, deg=deg,
                     ones_v=ones_v, dacc=dacc, dsem=dsem)
    else:
        def body(val, srcr, dstr, zer, out, idx_s, idx_d, ring, acc, gsem, ssem):
            _sc_body(ring_n, fire, val, srcr, dstr, zer, out, idx_s, idx_d,
                     ring, acc, gsem, ssem)

    return pl.kernel(body, out_type=out_type, mesh=mesh, scratch_types=scratch,
                     compiler_params=pltpu.CompilerParams(use_tc_tiling_on_sc=False))


def _layer(a0, a1, d0, d1, w):
    def body(a0r, a1r, d0r, d1r, wr, o_ref):
        degc = jnp.maximum((d0r[...] + d1r[...])[:, 0:1], 1.0)
        agg = (a0r[...] + a1r[...]) / degc
        o_ref[...] = jnp.maximum(
            jnp.dot(agg, wr[...], preferred_element_type=jnp.float32), 0.0)

    return pl.pallas_call(
        body,
        out_shape=jax.ShapeDtypeStruct((NPAD, w.shape[1]), jnp.float32),
    )(a0, a1, d0, d1, w)


def _layer_cat(a0, a1, b0, b1, d0, d1, w):
    def body(a0r, a1r, b0r, b1r, d0r, d1r, wr, o_ref):
        degc = jnp.maximum((d0r[...] + d1r[...])[:, 0:1], 1.0)
        agg = jnp.concatenate(
            [(a0r[...] + a1r[...]) / degc, (b0r[...] + b1r[...]) / degc], axis=1)
        o_ref[...] = jnp.maximum(
            jnp.dot(agg, wr[...], preferred_element_type=jnp.float32), 0.0)

    return pl.pallas_call(
        body,
        out_shape=jax.ShapeDtypeStruct((NPAD, w.shape[1]), jnp.float32),
    )(a0, a1, b0, b1, d0, d1, w)


def _score_mm(a0, a1, d0, d1, wpad):
    def body(a0r, a1r, d0r, d1r, wr, o_ref):
        degc = jnp.maximum((d0r[...] + d1r[...])[:, 0:1], 1.0)
        agg = (a0r[...] + a1r[...]) / degc
        o_ref[...] = jnp.tanh(
            jnp.dot(agg, wr[...], preferred_element_type=jnp.float32))

    return pl.pallas_call(
        body,
        out_shape=jax.ShapeDtypeStruct((NPAD, 8), jnp.float32),
    )(a0, a1, d0, d1, wpad)


SRW = NPAD // 128


def _select(sc2d):
    def body(s_ref, w_ref):
        score = s_ref[...]
        row = lax.broadcasted_iota(jnp.int32, (SRW, 128), 0)
        col = lax.broadcasted_iota(jnp.int32, (SRW, 128), 1)
        idxm = row * 128 + col
        valid = idxm < N
        bits = lax.bitcast_convert_type(score, jnp.int32)
        skey = jnp.where(bits >= 0, bits, jnp.invert(bits) ^ INT_MIN)
        skey = jnp.where(valid, skey, INT_MIN)

        def sbody(t, pref):
            cand = pref | (jnp.int32(1) << (31 - t))
            cnt = jnp.sum((skey >= (cand ^ INT_MIN)).astype(jnp.int32))
            return jnp.where(cnt >= K, cand, pref)

        pref = lax.fori_loop(0, 32, sbody, jnp.int32(0))
        vs = pref ^ INT_MIN
        cgt = jnp.sum((skey > vs).astype(jnp.int32))
        m = K - cgt
        tie = skey == vs

        def s2body(t, ans):
            cand = ans | (jnp.int32(1) << (13 - t))
            cnt = jnp.sum((tie & (idxm < cand)).astype(jnp.int32))
            return jnp.where(cnt < m, cand, ans)

        ans = lax.fori_loop(0, 14, s2body, jnp.int32(0))
        sel = (skey > vs) | (tie & (idxm <= ans))
        w_ref[...] = jnp.where(sel, score, 0.0)

    return pl.pallas_call(
        body,
        out_shape=jax.ShapeDtypeStruct((SRW, 128), jnp.float32),
    )(sc2d)


def _readout(wcol, gi8, h2):
    def body(w_ref, gi_ref, h2_ref, o_ref):
        hm = h2_ref[...] * w_ref[...]
        giota = lax.broadcasted_iota(jnp.int32, (1, G), 1)
        oh = (gi_ref[...][:, 0:1] == giota).astype(jnp.float32)
        o_ref[...] = lax.dot_general(
            oh, hm, (((0,), (0,)), ((), ())),
            preferred_element_type=jnp.float32)

    return pl.pallas_call(
        body,
        out_shape=jax.ShapeDtypeStruct((G, DH), jnp.float32),
    )(wcol, gi8, h2)


def kernel(adjacency, graph_indicator, eeg, eye, au, W1, W2, w_score):
    src = adjacency[0]
    dst = adjacency[1]
    x = jnp.pad(eeg.reshape(-1, D_IN), ((0, NPAD - N), (0, 0)))
    xa = x[:, :DH]
    xb = x[:, DH:]

    pad = EPAD - E
    srcr = jnp.concatenate([src, jnp.zeros((pad,), jnp.int32)]).reshape(NW, CH, C)
    dstr = jnp.concatenate([dst, jnp.full((pad,), N, jnp.int32)]).reshape(NW, CH, C)
    zer64 = jnp.zeros((NPAD, DH), jnp.float32)
    zer8 = jnp.zeros((NPAD, 8), jnp.float32)
    ones8 = jnp.ones((C, 8), jnp.float32)

    sum1a, degp = _make_sc_segsum(DH, True)(xa, srcr, dstr, zer64, zer8, ones8)
    sum1b = _make_sc_segsum(DH, False)(xb, srcr, dstr, zer64)
    h1 = _layer_cat(sum1a[0], sum1a[1], sum1b[0], sum1b[1],
                    degp[0], degp[1], W1)
    sum2 = _make_sc_segsum(DH, False)(h1, srcr, dstr, zer64)
    h2 = _layer(sum2[0], sum2[1], degp[0], degp[1], W2)
    sum3 = _make_sc_segsum(DH, False)(h2, srcr, dstr, zer64)
    wpad = jnp.pad(w_score, ((0, 0), (0, 7)))
    sc8 = _score_mm(sum3[0], sum3[1], degp[0], degp[1], wpad)
    w2d = _select(sc8[:, 0].reshape(SRW, 128))
    wcol = w2d.reshape(NPAD, 1)
    gip = jnp.pad(graph_indicator, (0, NPAD - N), constant_values=G)
    gi8 = jnp.broadcast_to(gip[:, None], (NPAD, 8))
    eeg_out = _readout(wcol, gi8, h2)
    return (eeg_out, eye, au)


# speedup vs baseline: 4.2860x; 1.0304x over previous
import functools

import numpy as np
import jax
import jax.numpy as jnp
from jax import lax
from jax.experimental import pallas as pl
from jax.experimental.pallas import tpu as pltpu
from jax.experimental.pallas import tpu_sc as plsc

N = 10000
E = 320000
D_IN = 128
DH = 64
G = 64
K = 5000

NW = 32
C = 128
CH = 80
EPAD = NW * CH * C
NPAD = 10112
RPW = NPAD // 16
INT_MIN = np.int32(-(2 ** 31))


def _sc_body(ring_n, fire, val, srcr, dstr, zer, out, idx_s, idx_d, ring, acc,
             gsem, ssem, zer8=None, ones8=None, deg=None, ones_v=None,
             dacc=None, dsem=None):
    c = lax.axis_index("c")
    s = lax.axis_index("s")
    wid = s * 2 + c
    pltpu.sync_copy(srcr.at[wid], idx_s)
    pltpu.sync_copy(dstr.at[wid], idx_d)
    for b in range(fire):
        pltpu.async_copy(val.at[idx_s.at[b]], ring.at[b], gsem.at[b])
    pltpu.sync_copy(zer.at[pl.ds(s * RPW, RPW)], acc.at[pl.ds(s * RPW, RPW)])
    if dacc is not None:
        pltpu.sync_copy(zer8.at[pl.ds(s * RPW, RPW)], dacc.at[pl.ds(s * RPW, RPW)])
        pltpu.sync_copy(ones8, ones_v)
    plsc.subcore_barrier()

    @pl.loop(0, CH, step=ring_n)
    def _outer(jo):
        for b in range(ring_n):
            j = jo + b
            pltpu.make_async_copy(val.at[idx_s.at[j]], ring.at[b], gsem.at[b]).wait()
            pltpu.async_copy(ring.at[b], acc.at[idx_d.at[j]], ssem.at[b], add=True)
            if dacc is not None:
                pltpu.async_copy(ones_v, dacc.at[idx_d.at[j]], dsem.at[b], add=True)
            fj = j + fire
            bf = (b + fire) % ring_n

            @pl.when((fj >= ring_n) & (fj < CH))
            def _drain():
                pj = fj - ring_n
                pltpu.make_async_copy(ring.at[bf], acc.at[idx_d.at[pj]],
                                      ssem.at[bf]).wait()
                if dacc is not None:
                    pltpu.make_async_copy(ones_v, dacc.at[idx_d.at[pj]],
                                          dsem.at[bf]).wait()

            @pl.when(fj < CH)
            def _fire():
                pltpu.async_copy(val.at[idx_s.at[fj]], ring.at[bf], gsem.at[bf])

    for q in range(CH - ring_n, CH):
        bq = q % ring_n
        pltpu.make_async_copy(ring.at[bq], acc.at[idx_d.at[q]], ssem.at[bq]).wait()
        if dacc is not None:
            pltpu.make_async_copy(ones_v, dacc.at[idx_d.at[q]], dsem.at[bq]).wait()

    plsc.subcore_barrier()
    pltpu.sync_copy(acc.at[pl.ds(s * RPW, RPW)], out.at[c, pl.ds(s * RPW, RPW)])
    if dacc is not None:
        pltpu.sync_copy(dacc.at[pl.ds(s * RPW, RPW)], deg.at[c, pl.ds(s * RPW, RPW)])


@functools.cache
def _make_sc_segsum(D, with_deg):
    mesh = plsc.VectorSubcoreMesh(core_axis_name="c", subcore_axis_name="s")
    ring_n = 5 if with_deg else 8
    fire = ring_n // 2
    out_type = jax.ShapeDtypeStruct((2, NPAD, D), jnp.float32)
    scratch = [
        pltpu.VMEM((CH, C), jnp.int32),
        pltpu.VMEM((CH, C), jnp.int32),
        pltpu.VMEM((ring_n, C, D), jnp.float32),
        pltpu.VMEM_SHARED((NPAD, D), jnp.float32),
        pltpu.SemaphoreType.DMA((ring_n,)),
        pltpu.SemaphoreType.DMA((ring_n,)),
    ]
    if with_deg:
        out_type = (out_type, jax.ShapeDtypeStruct((2, NPAD, 8), jnp.float32))
        scratch += [
            pltpu.VMEM((C, 8), jnp.float32),
            pltpu.VMEM_SHARED((NPAD, 8), jnp.float32),
            pltpu.SemaphoreType.DMA((ring_n,)),
        ]

        def body(val, srcr, dstr, zer, zer8, ones8, out, deg,
                 idx_s, idx_d, ring, acc, gsem, ssem, ones_v, dacc, dsem):
            _sc_body(ring_n, fire, val, srcr, dstr, zer, out, idx_s, idx_d,
                     ring, acc, gsem, ssem, zer8=zer8, ones8=ones8, deg=deg,
                     ones_v=ones_v, dacc=dacc, dsem=dsem)
    else:
        def body(val, srcr, dstr, zer, out, idx_s, idx_d, ring, acc, gsem, ssem):
            _sc_body(ring_n, fire, val, srcr, dstr, zer, out, idx_s, idx_d,
                     ring, acc, gsem, ssem)

    return pl.kernel(body, out_type=out_type, mesh=mesh, scratch_types=scratch,
                     compiler_params=pltpu.CompilerParams(use_tc_tiling_on_sc=False))


def _layer(a0, a1, d0, d1, w):
    def body(a0r, a1r, d0r, d1r, wr, o_ref):
        degc = jnp.maximum((d0r[...] + d1r[...])[:, 0:1], 1.0)
        agg = (a0r[...] + a1r[...]) / degc
        o_ref[...] = jnp.maximum(
            jnp.dot(agg, wr[...], preferred_element_type=jnp.float32), 0.0)

    return pl.pallas_call(
        body,
        out_shape=jax.ShapeDtypeStruct((NPAD, w.shape[1]), jnp.float32),
    )(a0, a1, d0, d1, w)


def _layer_cat(a0, a1, b0, b1, d0, d1, w):
    def body(a0r, a1r, b0r, b1r, d0r, d1r, wr, o_ref):
        degc = jnp.maximum((d0r[...] + d1r[...])[:, 0:1], 1.0)
        agg = jnp.concatenate(
            [(a0r[...] + a1r[...]) / degc, (b0r[...] + b1r[...]) / degc], axis=1)
        o_ref[...] = jnp.maximum(
            jnp.dot(agg, wr[...], preferred_element_type=jnp.float32), 0.0)

    return pl.pallas_call(
        body,
        out_shape=jax.ShapeDtypeStruct((NPAD, w.shape[1]), jnp.float32),
    )(a0, a1, b0, b1, d0, d1, w)


def _score_mm(a0, a1, d0, d1, wpad):
    def body(a0r, a1r, d0r, d1r, wr, o_ref):
        degc = jnp.maximum((d0r[...] + d1r[...])[:, 0:1], 1.0)
        agg = (a0r[...] + a1r[...]) / degc
        o_ref[...] = jnp.tanh(
            jnp.dot(agg, wr[...], preferred_element_type=jnp.float32))

    return pl.pallas_call(
        body,
        out_shape=jax.ShapeDtypeStruct((NPAD, 8), jnp.float32),
    )(a0, a1, d0, d1, wpad)


SRW = NPAD // 128


def _select(sc2d):
    def body(s_ref, w_ref):
        score = s_ref[...]
        row = lax.broadcasted_iota(jnp.int32, (SRW, 128), 0)
        col = lax.broadcasted_iota(jnp.int32, (SRW, 128), 1)
        idxm = row * 128 + col
        valid = idxm < N
        bits = lax.bitcast_convert_type(score, jnp.int32)
        skey = jnp.where(bits >= 0, bits, jnp.invert(bits) ^ INT_MIN)
        skey = jnp.where(valid, skey, INT_MIN)

        def sbody(t, pref):
            cand = pref | (jnp.int32(1) << (31 - t))
            cnt = jnp.sum((skey >= (cand ^ INT_MIN)).astype(jnp.int32))
            return jnp.where(cnt >= K, cand, pref)

        pref = lax.fori_loop(0, 32, sbody, jnp.int32(0))
        vs = pref ^ INT_MIN
        cgt = jnp.sum((skey > vs).astype(jnp.int32))
        m = K - cgt
        tie = skey == vs

        def s2body(t, ans):
            cand = ans | (jnp.int32(1) << (13 - t))
            cnt = jnp.sum((tie & (idxm < cand)).astype(jnp.int32))
            return jnp.where(cnt < m, cand, ans)

        ans = lax.fori_loop(0, 14, s2body, jnp.int32(0))
        sel = (skey > vs) | (tie & (idxm <= ans))
        w_ref[...] = jnp.where(sel, score, 0.0)

    return pl.pallas_call(
        body,
        out_shape=jax.ShapeDtypeStruct((SRW, 128), jnp.float32),
    )(sc2d)


def _readout(wcol, gi8, h2):
    def body(w_ref, gi_ref, h2_ref, o_ref):
        hm = h2_ref[...] * w_ref[...]
        giota = lax.broadcasted_iota(jnp.int32, (1, G), 1)
        oh = (gi_ref[...][:, 0:1] == giota).astype(jnp.float32)
        o_ref[...] = lax.dot_general(
            oh, hm, (((0,), (0,)), ((), ())),
            preferred_element_type=jnp.float32)

    return pl.pallas_call(
        body,
        out_shape=jax.ShapeDtypeStruct((G, DH), jnp.float32),
    )(wcol, gi8, h2)


def kernel(adjacency, graph_indicator, eeg, eye, au, W1, W2, w_score):
    src = adjacency[0]
    dst = adjacency[1]
    x = eeg.reshape(-1, D_IN)
    xa = x[:, :DH]
    xb = x[:, DH:]

    pad = EPAD - E
    srcr = jnp.concatenate([src, jnp.zeros((pad,), jnp.int32)]).reshape(NW, CH, C)
    dstr = jnp.concatenate([dst, jnp.full((pad,), N, jnp.int32)]).reshape(NW, CH, C)
    zer64 = jnp.zeros((NPAD, DH), jnp.float32)
    zer8 = jnp.zeros((NPAD, 8), jnp.float32)
    ones8 = jnp.ones((C, 8), jnp.float32)

    sum1a, degp = _make_sc_segsum(DH, True)(xa, srcr, dstr, zer64, zer8, ones8)
    sum1b = _make_sc_segsum(DH, False)(xb, srcr, dstr, zer64)
    h1 = _layer_cat(sum1a[0], sum1a[1], sum1b[0], sum1b[1],
                    degp[0], degp[1], W1)
    sum2 = _make_sc_segsum(DH, False)(h1, srcr, dstr, zer64)
    h2 = _layer(sum2[0], sum2[1], degp[0], degp[1], W2)
    sum3 = _make_sc_segsum(DH, False)(h2, srcr, dstr, zer64)
    wpad = jnp.pad(w_score, ((0, 0), (0, 7)))
    sc8 = _score_mm(sum3[0], sum3[1], degp[0], degp[1], wpad)
    w2d = _select(sc8[:, 0].reshape(SRW, 128))
    wcol = w2d.reshape(NPAD, 1)
    gip = jnp.pad(graph_indicator, (0, NPAD - N), constant_values=G)
    gi8 = jnp.broadcast_to(gip[:, None], (NPAD, 8))
    eeg_out = _readout(wcol, gi8, h2)
    return (eeg_out, eye, au)


# speedup vs baseline: 4.6187x; 1.0776x over previous
import functools

import numpy as np
import jax
import jax.numpy as jnp
from jax import lax
from jax.experimental import pallas as pl
from jax.experimental.pallas import tpu as pltpu
from jax.experimental.pallas import tpu_sc as plsc

N = 10000
E = 320000
D_IN = 128
DH = 64
G = 64
K = 5000

NW = 32
C = 128
CH0 = 128
CH1 = 32
CPOOL = 16 * CH0 + 15 * CH1 + CH0
EPAD = CPOOL * C
NPAD = 10112
RPW = NPAD // 16
RING = 4
FIRE = 2
INT_MIN = np.int32(-(2 ** 31))


def _sc_body(val, srcr, dstr, zer, out, idx_s, idx_d, ring, acc,
             gsem, ssem, zer8=None, ones8=None, deg=None, ones_v=None,
             dacc=None, dsem=None):
    c = lax.axis_index("c")
    s = lax.axis_index("s")
    my_ch = jnp.where(c == 0, CH0, CH1)
    start = jnp.where(c == 0, s * CH0, 16 * CH0 + s * CH1)
    pltpu.sync_copy(srcr.at[pl.ds(start, CH0)], idx_s)
    pltpu.sync_copy(dstr.at[pl.ds(start, CH0)], idx_d)
    for b in range(FIRE):
        pltpu.async_copy(val.at[idx_s.at[b]], ring.at[b], gsem.at[b])
    pltpu.sync_copy(zer.at[pl.ds(s * RPW, RPW)], acc.at[pl.ds(s * RPW, RPW)])
    if dacc is not None:
        pltpu.sync_copy(zer8.at[pl.ds(s * RPW, RPW)], dacc.at[pl.ds(s * RPW, RPW)])
        pltpu.sync_copy(ones8, ones_v)
    plsc.subcore_barrier()

    @pl.loop(0, my_ch, step=RING)
    def _outer(jo):
        for b in range(RING):
            j = jo + b
            pltpu.make_async_copy(val.at[idx_s.at[j]], ring.at[b], gsem.at[b]).wait()
            pltpu.async_copy(ring.at[b], acc.at[idx_d.at[j]], ssem.at[b], add=True)
            if dacc is not None:
                pltpu.async_copy(ones_v, dacc.at[idx_d.at[j]], dsem.at[b], add=True)
            fj = j + FIRE
            bf = (b + FIRE) % RING

            @pl.when((fj >= RING) & (fj < my_ch))
            def _drain():
                pj = fj - RING
                pltpu.make_async_copy(ring.at[bf], acc.at[idx_d.at[pj]],
                                      ssem.at[bf]).wait()
                if dacc is not None:
                    pltpu.make_async_copy(ones_v, dacc.at[idx_d.at[pj]],
                                          dsem.at[bf]).wait()

            @pl.when(fj < my_ch)
            def _fire():
                pltpu.async_copy(val.at[idx_s.at[fj]], ring.at[bf], gsem.at[bf])

    for b in range(RING):
        q = my_ch - RING + b
        pltpu.make_async_copy(ring.at[b], acc.at[idx_d.at[q]], ssem.at[b]).wait()
        if dacc is not None:
            pltpu.make_async_copy(ones_v, dacc.at[idx_d.at[q]], dsem.at[b]).wait()

    plsc.subcore_barrier()
    pltpu.sync_copy(acc.at[pl.ds(s * RPW, RPW)], out.at[c, pl.ds(s * RPW, RPW)])
    if dacc is not None:
        pltpu.sync_copy(dacc.at[pl.ds(s * RPW, RPW)], deg.at[c, pl.ds(s * RPW, RPW)])


@functools.cache
def _make_sc_segsum(D, with_deg):
    mesh = plsc.VectorSubcoreMesh(core_axis_name="c", subcore_axis_name="s")
    out_type = jax.ShapeDtypeStruct((2, NPAD, D), jnp.float32)
    scratch = [
        pltpu.VMEM((CH0, C), jnp.int32),
        pltpu.VMEM((CH0, C), jnp.int32),
        pltpu.VMEM((RING, C, D), jnp.float32),
        pltpu.VMEM_SHARED((NPAD, D), jnp.float32),
        pltpu.SemaphoreType.DMA((RING,)),
        pltpu.SemaphoreType.DMA((RING,)),
    ]
    if with_deg:
        out_type = (out_type, jax.ShapeDtypeStruct((2, NPAD, 8), jnp.float32))
        scratch += [
            pltpu.VMEM((C, 8), jnp.float32),
            pltpu.VMEM_SHARED((NPAD, 8), jnp.float32),
            pltpu.SemaphoreType.DMA((RING,)),
        ]

        def body(val, srcr, dstr, zer, zer8, ones8, out, deg,
                 idx_s, idx_d, ring, acc, gsem, ssem, ones_v, dacc, dsem):
            _sc_body(val, srcr, dstr, zer, out, idx_s, idx_d,
                     ring, acc, gsem, ssem, zer8=zer8, ones8=ones8, deg=deg,
                     ones_v=ones_v, dacc=dacc, dsem=dsem)
    else:
        def body(val, srcr, dstr, zer, out, idx_s, idx_d, ring, acc, gsem, ssem):
            _sc_body(val, srcr, dstr, zer, out, idx_s, idx_d,
                     ring, acc, gsem, ssem)

    return pl.kernel(body, out_type=out_type, mesh=mesh, scratch_types=scratch,
                     compiler_params=pltpu.CompilerParams(use_tc_tiling_on_sc=False))


def _layer(a0, a1, d0, d1, w):
    def body(a0r, a1r, d0r, d1r, wr, o_ref):
        degc = jnp.maximum((d0r[...] + d1r[...])[:, 0:1], 1.0)
        agg = (a0r[...] + a1r[...]) / degc
        o_ref[...] = jnp.maximum(
            jnp.dot(agg, wr[...], preferred_element_type=jnp.float32), 0.0)

    return pl.pallas_call(
        body,
        out_shape=jax.ShapeDtypeStruct((NPAD, w.shape[1]), jnp.float32),
    )(a0, a1, d0, d1, w)


def _layer_cat(a0, a1, b0, b1, d0, d1, w):
    def body(a0r, a1r, b0r, b1r, d0r, d1r, wr, o_ref):
        degc = jnp.maximum((d0r[...] + d1r[...])[:, 0:1], 1.0)
        agg = jnp.concatenate(
            [(a0r[...] + a1r[...]) / degc, (b0r[...] + b1r[...]) / degc], axis=1)
        o_ref[...] = jnp.maximum(
            jnp.dot(agg, wr[...], preferred_element_type=jnp.float32), 0.0)

    return pl.pallas_call(
        body,
        out_shape=jax.ShapeDtypeStruct((NPAD, w.shape[1]), jnp.float32),
    )(a0, a1, b0, b1, d0, d1, w)


def _score_mm(a0, a1, d0, d1, wpad):
    def body(a0r, a1r, d0r, d1r, wr, o_ref):
        degc = jnp.maximum((d0r[...] + d1r[...])[:, 0:1], 1.0)
        agg = (a0r[...] + a1r[...]) / degc
        o_ref[...] = jnp.tanh(
            jnp.dot(agg, wr[...], preferred_element_type=jnp.float32))

    return pl.pallas_call(
        body,
        out_shape=jax.ShapeDtypeStruct((NPAD, 8), jnp.float32),
    )(a0, a1, d0, d1, wpad)


SRW = NPAD // 128


def _select(sc2d):
    def body(s_ref, w_ref):
        score = s_ref[...]
        row = lax.broadcasted_iota(jnp.int32, (SRW, 128), 0)
        col = lax.broadcasted_iota(jnp.int32, (SRW, 128), 1)
        idxm = row * 128 + col
        valid = idxm < N
        bits = lax.bitcast_convert_type(score, jnp.int32)
        skey = jnp.where(bits >= 0, bits, jnp.invert(bits) ^ INT_MIN)
        skey = jnp.where(valid, skey, INT_MIN)

        def sbody(t, pref):
            cand = pref | (jnp.int32(1) << (31 - t))
            cnt = jnp.sum((skey >= (cand ^ INT_MIN)).astype(jnp.int32))
            return jnp.where(cnt >= K, cand, pref)

        pref = lax.fori_loop(0, 32, sbody, jnp.int32(0))
        vs = pref ^ INT_MIN
        cgt = jnp.sum((skey > vs).astype(jnp.int32))
        m = K - cgt
        tie = skey == vs

        def s2body(t, ans):
            cand = ans | (jnp.int32(1) << (13 - t))
            cnt = jnp.sum((tie & (idxm < cand)).astype(jnp.int32))
            return jnp.where(cnt < m, cand, ans)

        ans = lax.fori_loop(0, 14, s2body, jnp.int32(0))
        sel = (skey > vs) | (tie & (idxm <= ans))
        w_ref[...] = jnp.where(sel, score, 0.0)

    return pl.pallas_call(
        body,
        out_shape=jax.ShapeDtypeStruct((SRW, 128), jnp.float32),
    )(sc2d)


def _readout(wcol, gi8, h2):
    def body(w_ref, gi_ref, h2_ref, o_ref):
        hm = h2_ref[...] * w_ref[...]
        giota = lax.broadcasted_iota(jnp.int32, (1, G), 1)
        oh = (gi_ref[...][:, 0:1] == giota).astype(jnp.float32)
        o_ref[...] = lax.dot_general(
            oh, hm, (((0,), (0,)), ((), ())),
            preferred_element_type=jnp.float32)

    return pl.pallas_call(
        body,
        out_shape=jax.ShapeDtypeStruct((G, DH), jnp.float32),
    )(wcol, gi8, h2)


def kernel(adjacency, graph_indicator, eeg, eye, au, W1, W2, w_score):
    src = adjacency[0]
    dst = adjacency[1]
    x = eeg.reshape(-1, D_IN)
    xa = x[:, :DH]
    xb = x[:, DH:]

    pad = EPAD - E
    srcr = jnp.concatenate([src, jnp.zeros((pad,), jnp.int32)]).reshape(CPOOL, C)
    dstr = jnp.concatenate([dst, jnp.full((pad,), N, jnp.int32)]).reshape(CPOOL, C)
    zer64 = jnp.zeros((NPAD, DH), jnp.float32)
    zer8 = jnp.zeros((NPAD, 8), jnp.float32)
    ones8 = jnp.ones((C, 8), jnp.float32)

    sum1a, degp = _make_sc_segsum(DH, True)(xa, srcr, dstr, zer64, zer8, ones8)
    sum1b = _make_sc_segsum(DH, False)(xb, srcr, dstr, zer64)
    h1 = _layer_cat(sum1a[0], sum1a[1], sum1b[0], sum1b[1],
                    degp[0], degp[1], W1)
    sum2 = _make_sc_segsum(DH, False)(h1, srcr, dstr, zer64)
    h2 = _layer(sum2[0], sum2[1], degp[0], degp[1], W2)
    sum3 = _make_sc_segsum(DH, False)(h2, srcr, dstr, zer64)
    wpad = jnp.pad(w_score, ((0, 0), (0, 7)))
    sc8 = _score_mm(sum3[0], sum3[1], degp[0], degp[1], wpad)
    w2d = _select(sc8[:, 0].reshape(SRW, 128))
    wcol = w2d.reshape(NPAD, 1)
    gip = jnp.pad(graph_indicator, (0, NPAD - N), constant_values=G)
    gi8 = jnp.broadcast_to(gip[:, None], (NPAD, 8))
    eeg_out = _readout(wcol, gi8, h2)
    return (eeg_out, eye, au)
